# Initial kernel scaffold; baseline (speedup 1.0000x reference)
import functools

import jax
import jax.numpy as jnp
from jax import lax
from jax.experimental import pallas as pl
from jax.experimental.pallas import tpu as pltpu
from jax.experimental.pallas import tpu_sc as plsc

_NC = 2
_NS = 16
_L = 16
_K = 128
_F = 128


def _leaky(v):
    return jnp.where(v >= 0, v, 0.01 * v)


def _mesh():
    return plsc.VectorSubcoreMesh(
        core_axis_name="c", subcore_axis_name="s",
        num_cores=_NC, num_subcores=_NS)


def _deg_call(dst2, w2, np_):
    rows2d = dst2.shape[0]
    per_core = rows2d // _NC
    per_tile = per_core // _NS
    stripe = np_ // _NS

    @functools.partial(
        pl.kernel,
        out_type=jax.ShapeDtypeStruct((_NC, np_), jnp.float32),
        mesh=_mesh(),
        scratch_types=[
            pltpu.VMEM((per_tile, _K), jnp.int32),
            pltpu.VMEM((per_tile, _K), jnp.float32),
            pltpu.VMEM((stripe,), jnp.float32),
            pltpu.VMEM_SHARED((np_,), jnp.float32),
        ],
    )
    def deg_kernel(dst_hbm, w_hbm, out_hbm, dstb, wb, zb, dacc):
        c = lax.axis_index("c")
        s = lax.axis_index("s")
        base = c * per_core + s * per_tile
        pltpu.sync_copy(dst_hbm.at[pl.ds(base, per_tile)], dstb)
        pltpu.sync_copy(w_hbm.at[pl.ds(base, per_tile)], wb)

        def zero_body(i, carry):
            zb[pl.ds(i * _L, _L)] = jnp.zeros((_L,), jnp.float32)
            return carry

        lax.fori_loop(0, stripe // _L, zero_body, 0)
        pltpu.sync_copy(zb, dacc.at[pl.ds(s * stripe, stripe)])
        plsc.subcore_barrier()

        def scat_body(j, carry):
            pltpu.sync_copy(wb.at[j], dacc.at[dstb.at[j]], add=True)
            return carry

        lax.fori_loop(0, per_tile, scat_body, 0)
        plsc.subcore_barrier()
        pltpu.sync_copy(dacc.at[pl.ds(s * stripe, stripe)],
                        out_hbm.at[c, pl.ds(s * stripe, stripe)])

    return deg_kernel(dst2, w2)


def _agg_call(c_lo, c_hi, src2, dst2, w2, np_):
    rows2d = src2.shape[0]
    per_tile = rows2d // _NS
    stripe = np_ // _NS

    @functools.partial(
        pl.kernel,
        out_type=[jax.ShapeDtypeStruct((np_, _F), jnp.float32),
                  jax.ShapeDtypeStruct((np_, _F), jnp.float32)],
        mesh=_mesh(),
        scratch_types=[
            pltpu.VMEM((per_tile, _K), jnp.int32),
            pltpu.VMEM((per_tile, _K), jnp.int32),
            pltpu.VMEM((per_tile, _K), jnp.float32),
            pltpu.VMEM((_K, _F), jnp.float32),
            pltpu.VMEM_SHARED((np_, _F), jnp.float32),
            pltpu.SemaphoreType.DMA,
        ],
    )
    def agg_kernel(clo_hbm, chi_hbm, src_hbm, dst_hbm, w_hbm,
                   olo_hbm, ohi_hbm, srcb, dstb, wb, rows, acc, sem):
        c = lax.axis_index("c")
        s = lax.axis_index("s")
        pltpu.sync_copy(src_hbm.at[pl.ds(s * per_tile, per_tile)], srcb)
        pltpu.sync_copy(dst_hbm.at[pl.ds(s * per_tile, per_tile)], dstb)
        pltpu.sync_copy(w_hbm.at[pl.ds(s * per_tile, per_tile)], wb)

        def zero_row(i, carry):
            for g in range(_F // _L):
                rows[i, pl.ds(g * _L, _L)] = jnp.zeros((_L,), jnp.float32)
            return carry

        lax.fori_loop(0, _K, zero_row, 0)
        for b in range(stripe // _K):
            pltpu.sync_copy(rows, acc.at[pl.ds(s * stripe + b * _K, _K)])
        plsc.subcore_barrier()

        def run(chbm, ohbm):
            def chunk(j, carry):
                pltpu.async_copy(chbm.at[srcb.at[j]], rows, sem).wait()

                def scale(k, kc):
                    wk = wb[j, k]
                    for g in range(_F // _L):
                        sl = pl.ds(g * _L, _L)
                        rows[k, sl] = rows[k, sl] * wk
                    return kc

                lax.fori_loop(0, _K, scale, 0)
                pltpu.sync_copy(rows, acc.at[dstb.at[j]], add=True)
                return carry

            lax.fori_loop(0, per_tile, chunk, 0)
            plsc.subcore_barrier()
            pltpu.sync_copy(acc.at[pl.ds(s * stripe, stripe)],
                            ohbm.at[pl.ds(s * stripe, stripe)])

        @pl.when(c == 0)
        def _():
            run(clo_hbm, olo_hbm)

        @pl.when(c == 1)
        def _():
            run(chi_hbm, ohi_hbm)

    return agg_kernel(c_lo, c_hi, src2, dst2, w2)


def _tc(body, out_shape, *args):
    return pl.pallas_call(
        body,
        out_shape=out_shape,
        compiler_params=pltpu.CompilerParams(
            vmem_limit_bytes=128 * 1024 * 1024),
    )(*args)


def _dinv_from(degT_ref):
    degT = degT_ref[...]
    deg = degT[:, 0:1] + degT[:, 1:2] + 1.0
    return jnp.where(deg > 0, lax.rsqrt(deg), 0.0)


def _t1_body(x_ref, w1_ref, degT_ref, clo_ref, chi_ref):
    dinv = _dinv_from(degT_ref)
    h = jnp.dot(x_ref[...], w1_ref[...], preferred_element_type=jnp.float32)
    cmat = h * dinv
    clo_ref[...] = cmat[:, :_F]
    chi_ref[...] = cmat[:, _F:]


def _bn(z, g, be, n):
    m = jnp.mean(z, axis=0, keepdims=True)
    v = jnp.mean((z - m) ** 2, axis=0, keepdims=True)
    return (z - m) * lax.rsqrt(v + 1e-5) * g.reshape(1, n) + be.reshape(1, n)


def _t2_body(slo_ref, shi_ref, clo_ref, chi_ref, degT_ref, b1_ref, g1_ref,
             be1_ref, w2_ref, h1_ref, c2lo_ref, c2hi_ref, *, n, np_):
    ch = 2 * _F
    dinv = _dinv_from(degT_ref)[:n]
    s1 = jnp.concatenate([slo_ref[...][:n], shi_ref[...][:n]], axis=1)
    c1 = jnp.concatenate([clo_ref[...][:n], chi_ref[...][:n]], axis=1)
    g1out = dinv * (s1 + c1) + b1_ref[...].reshape(1, ch)
    h1 = _leaky(_bn(g1out, g1_ref[...], be1_ref[...], ch))
    h1_ref[...] = h1
    c2 = dinv * jnp.dot(h1, w2_ref[...], preferred_element_type=jnp.float32)
    c2p = jnp.concatenate(
        [c2, jnp.zeros((np_ - n, ch), jnp.float32)], axis=0)
    c2lo_ref[...] = c2p[:, :_F]
    c2hi_ref[...] = c2p[:, _F:]


def _t3a_body(slo_ref, shi_ref, clo_ref, chi_ref, degT_ref, b2_ref, g2_ref,
              be2_ref, h1_ref, h2_ref, *, n):
    ch = 2 * _F
    dinv = _dinv_from(degT_ref)[:n]
    s2 = jnp.concatenate([slo_ref[...][:n], shi_ref[...][:n]], axis=1)
    c2 = jnp.concatenate([clo_ref[...][:n], chi_ref[...][:n]], axis=1)
    g2out = dinv * (s2 + c2) + b2_ref[...].reshape(1, ch)
    h2_ref[...] = _leaky(
        _bn(g2out, g2_ref[...], be2_ref[...], ch) + h1_ref[...])


def _t3b_body(h2_ref, batch_ref, wih_ref, whh_ref, bih_ref, bhh_ref,
              wl1_ref, bl1_ref, wl2_ref, bl2_ref, out_ref, *, n, nb, steps):
    ch = 2 * _F
    x = h2_ref[...]
    batch = batch_ref[...]
    gid = lax.broadcasted_iota(jnp.int32, (n, nb), 1)
    maskb = batch == gid
    wih = wih_ref[...]
    whh = whh_ref[...]
    bias = (bih_ref[...] + bhh_ref[...]).reshape(1, 4 * ch)
    h = jnp.zeros((nb, ch), jnp.float32)
    cst = jnp.zeros((nb, ch), jnp.float32)
    qs = jnp.zeros((nb, 2 * ch), jnp.float32)
    for _ in range(steps):
        gates = (lax.dot_general(qs, wih, (((1,), (1,)), ((), ())),
                                 preferred_element_type=jnp.float32)
                 + lax.dot_general(h, whh, (((1,), (1,)), ((), ())),
                                   preferred_element_type=jnp.float32)
                 + bias)
        i_g = jax.nn.sigmoid(gates[:, :ch])
        f_g = jax.nn.sigmoid(gates[:, ch:2 * ch])
        g_g = jnp.tanh(gates[:, 2 * ch:3 * ch])
        o_g = jax.nn.sigmoid(gates[:, 3 * ch:])
        cst = f_g * cst + i_g * g_g
        h = o_g * jnp.tanh(cst)
        p = lax.dot_general(x, h, (((1,), (1,)), ((), ())),
                            preferred_element_type=jnp.float32)
        pm = jnp.where(maskb, p, -1e30)
        m = jnp.max(pm, axis=0, keepdims=True)
        m0 = jnp.where(m > -1e29, m, 0.0)
        ex = jnp.where(maskb, jnp.exp(p - m0), 0.0)
        ssum = jnp.sum(ex, axis=0, keepdims=True)
        a = ex / (ssum + 1e-16)
        r = lax.dot_general(a, x, (((0,), (0,)), ((), ())),
                            preferred_element_type=jnp.float32)
        qs = jnp.concatenate([h, r], axis=1)
    o1 = _leaky(jnp.dot(qs, wl1_ref[...], preferred_element_type=jnp.float32)
                + bl1_ref[...].reshape(1, ch))
    out_ref[...] = (jnp.dot(o1, wl2_ref[...],
                            preferred_element_type=jnp.float32)
                    + bl2_ref[...].reshape(1, wl2_ref.shape[1]))


def kernel(x, edge_index, batch_idx, edge_weights, W1, b1, W2, b2, g1, be1,
           g2, be2, W_ih, W_hh, b_ih, b_hh, W_l1, b_l1, W_l2, b_l2):
    n, c_in = x.shape
    e = edge_index.shape[1]
    nb = 64
    steps = 10

    blk = _NC * _NS * _K
    ep = ((e + blk - 1) // blk) * blk
    ngr = _NS * _K
    np_ = ((n + ngr - 1) // ngr) * ngr

    pad = ep - e
    src = jnp.concatenate([edge_index[0], jnp.zeros((pad,), jnp.int32)])
    dst = jnp.concatenate([edge_index[1], jnp.zeros((pad,), jnp.int32)])
    w = jnp.concatenate([edge_weights, jnp.zeros((pad,), jnp.float32)])
    src2 = src.reshape(ep // _K, _K)
    dst2 = dst.reshape(ep // _K, _K)
    w2 = w.reshape(ep // _K, _K)
    xp = jnp.concatenate([x, jnp.zeros((np_ - n, c_in), x.dtype)])

    degp = _deg_call(dst2, w2, np_)
    degT = degp.T

    c1lo, c1hi = _tc(
        _t1_body,
        [jax.ShapeDtypeStruct((np_, _F), jnp.float32)] * 2,
        xp, W1, degT)

    s1lo, s1hi = _agg_call(c1lo, c1hi, src2, dst2, w2, np_)

    h1, c2lo, c2hi = _tc(
        functools.partial(_t2_body, n=n, np_=np_),
        [jax.ShapeDtypeStruct((n, 2 * _F), jnp.float32),
         jax.ShapeDtypeStruct((np_, _F), jnp.float32),
         jax.ShapeDtypeStruct((np_, _F), jnp.float32)],
        s1lo, s1hi, c1lo, c1hi, degT, b1, g1, be1, W2)

    s2lo, s2hi = _agg_call(c2lo, c2hi, src2, dst2, w2, np_)

    h2 = _tc(
        functools.partial(_t3a_body, n=n),
        jax.ShapeDtypeStruct((n, 2 * _F), jnp.float32),
        s2lo, s2hi, c2lo, c2hi, degT, b2, g2, be2, h1)

    out = _tc(
        functools.partial(_t3b_body, n=n, nb=nb, steps=steps),
        jax.ShapeDtypeStruct((nb, W_l2.shape[1]), jnp.float32),
        h2, batch_idx[:, None], W_ih, W_hh, b_ih, b_hh,
        W_l1, b_l1, W_l2, b_l2)
    return out


# speedup vs baseline: 7.7590x; 7.7590x over previous
import functools

import jax
import jax.numpy as jnp
from jax import lax
from jax.experimental import pallas as pl
from jax.experimental.pallas import tpu as pltpu
from jax.experimental.pallas import tpu_sc as plsc

_NC = 2
_NS = 16
_L = 16
_K = 128
_F = 128


def _leaky(v):
    return jnp.where(v >= 0, v, 0.01 * v)


def _mesh():
    return plsc.VectorSubcoreMesh(
        core_axis_name="c", subcore_axis_name="s",
        num_cores=_NC, num_subcores=_NS)


def _deg_call(dst2, w2, np_):
    rows2d = dst2.shape[0]
    per_core = rows2d // _NC
    per_tile = per_core // _NS
    stripe = np_ // _NS

    @functools.partial(
        pl.kernel,
        out_type=jax.ShapeDtypeStruct((_NC, np_), jnp.float32),
        mesh=_mesh(),
        scratch_types=[
            pltpu.VMEM((per_tile, _K), jnp.int32),
            pltpu.VMEM((per_tile, _K), jnp.float32),
            pltpu.VMEM((stripe,), jnp.float32),
            pltpu.VMEM_SHARED((np_,), jnp.float32),
        ],
    )
    def deg_kernel(dst_hbm, w_hbm, out_hbm, dstb, wb, zb, dacc):
        c = lax.axis_index("c")
        s = lax.axis_index("s")
        base = c * per_core + s * per_tile
        pltpu.sync_copy(dst_hbm.at[pl.ds(base, per_tile)], dstb)
        pltpu.sync_copy(w_hbm.at[pl.ds(base, per_tile)], wb)

        def zero_body(i, carry):
            zb[pl.ds(i * _L, _L)] = jnp.zeros((_L,), jnp.float32)
            return carry

        lax.fori_loop(0, stripe // _L, zero_body, 0)
        pltpu.sync_copy(zb, dacc.at[pl.ds(s * stripe, stripe)])
        plsc.subcore_barrier()

        def scat_body(j, carry):
            pltpu.sync_copy(wb.at[j], dacc.at[dstb.at[j]], add=True)
            return carry

        lax.fori_loop(0, per_tile, scat_body, 0)
        plsc.subcore_barrier()
        pltpu.sync_copy(dacc.at[pl.ds(s * stripe, stripe)],
                        out_hbm.at[c, pl.ds(s * stripe, stripe)])

    return deg_kernel(dst2, w2)


def _agg_call(c_lo, c_hi, src2, dst2, w2, np_):
    rows2d = src2.shape[0]
    per_tile = rows2d // _NS
    stripe = np_ // _NS

    @functools.partial(
        pl.kernel,
        out_type=[jax.ShapeDtypeStruct((np_, _F), jnp.float32),
                  jax.ShapeDtypeStruct((np_, _F), jnp.float32)],
        mesh=_mesh(),
        scratch_types=[
            pltpu.VMEM((8, _K), jnp.int32),
            pltpu.VMEM((8, _K), jnp.int32),
            pltpu.VMEM((8, _K), jnp.float32),
            pltpu.VMEM((_K, _F), jnp.float32),
            pltpu.VMEM_SHARED((np_, _F), jnp.float32),
            pltpu.SemaphoreType.DMA,
        ],
    )
    def agg_kernel(clo_hbm, chi_hbm, src_hbm, dst_hbm, w_hbm,
                   olo_hbm, ohi_hbm, srcb, dstb, wb, rows, acc, sem):
        c = lax.axis_index("c")
        s = lax.axis_index("s")

        def zero_row(i, carry):
            for g in range(_F // _L):
                rows[i, pl.ds(g * _L, _L)] = jnp.zeros((_L,), jnp.float32)
            return carry

        lax.fori_loop(0, _K, zero_row, 0)
        for b in range(stripe // _K):
            pltpu.sync_copy(rows, acc.at[pl.ds(s * stripe + b * _K, _K)])
        plsc.subcore_barrier()

        def run(chbm, ohbm):
            def group(jj, carry):
                base = s * per_tile + jj * 8
                pltpu.sync_copy(src_hbm.at[pl.ds(base, 8)], srcb)
                pltpu.sync_copy(dst_hbm.at[pl.ds(base, 8)], dstb)
                pltpu.sync_copy(w_hbm.at[pl.ds(base, 8)], wb)

                def chunk(j, jc):
                    pltpu.async_copy(chbm.at[srcb.at[j]], rows, sem).wait()

                    def scale(k16, kc):
                        wv = wb[j, pl.ds(k16 * _L, _L)]
                        for i in range(_L):
                            wk = wv[i]
                            for g in range(_F // _L):
                                sl = pl.ds(g * _L, _L)
                                rows[k16 * _L + i, sl] = (
                                    rows[k16 * _L + i, sl] * wk)
                        return kc

                    lax.fori_loop(0, _K // _L, scale, 0)
                    pltpu.sync_copy(rows, acc.at[dstb.at[j]], add=True)
                    return jc

                lax.fori_loop(0, 8, chunk, 0)
                return carry

            lax.fori_loop(0, per_tile // 8, group, 0)
            plsc.subcore_barrier()
            pltpu.sync_copy(acc.at[pl.ds(s * stripe, stripe)],
                            ohbm.at[pl.ds(s * stripe, stripe)])

        @pl.when(c == 0)
        def _():
            run(clo_hbm, olo_hbm)

        @pl.when(c == 1)
        def _():
            run(chi_hbm, ohi_hbm)

    return agg_kernel(c_lo, c_hi, src2, dst2, w2)


def _tc(body, out_shape, *args):
    return pl.pallas_call(
        body,
        out_shape=out_shape,
        compiler_params=pltpu.CompilerParams(
            vmem_limit_bytes=128 * 1024 * 1024),
    )(*args)


def _dinv_from(degT_ref):
    degT = degT_ref[...]
    deg = degT[:, 0:1] + degT[:, 1:2] + 1.0
    return jnp.where(deg > 0, lax.rsqrt(deg), 0.0)


def _t1_body(x_ref, w1_ref, degT_ref, clo_ref, chi_ref):
    dinv = _dinv_from(degT_ref)
    h = jnp.dot(x_ref[...], w1_ref[...], preferred_element_type=jnp.float32)
    cmat = h * dinv
    clo_ref[...] = cmat[:, :_F]
    chi_ref[...] = cmat[:, _F:]


def _bn(z, g, be, n):
    m = jnp.mean(z, axis=0, keepdims=True)
    v = jnp.mean((z - m) ** 2, axis=0, keepdims=True)
    return (z - m) * lax.rsqrt(v + 1e-5) * g.reshape(1, n) + be.reshape(1, n)


def _t2_body(slo_ref, shi_ref, clo_ref, chi_ref, degT_ref, b1_ref, g1_ref,
             be1_ref, w2_ref, h1_ref, c2lo_ref, c2hi_ref, *, n, np_):
    ch = 2 * _F
    dinv = _dinv_from(degT_ref)[:n]
    s1 = jnp.concatenate([slo_ref[...][:n], shi_ref[...][:n]], axis=1)
    c1 = jnp.concatenate([clo_ref[...][:n], chi_ref[...][:n]], axis=1)
    g1out = dinv * (s1 + c1) + b1_ref[...].reshape(1, ch)
    h1 = _leaky(_bn(g1out, g1_ref[...], be1_ref[...], ch))
    h1_ref[...] = h1
    c2 = dinv * jnp.dot(h1, w2_ref[...], preferred_element_type=jnp.float32)
    c2p = jnp.concatenate(
        [c2, jnp.zeros((np_ - n, ch), jnp.float32)], axis=0)
    c2lo_ref[...] = c2p[:, :_F]
    c2hi_ref[...] = c2p[:, _F:]


def _t3a_body(slo_ref, shi_ref, clo_ref, chi_ref, degT_ref, b2_ref, g2_ref,
              be2_ref, h1_ref, h2_ref, *, n):
    ch = 2 * _F
    dinv = _dinv_from(degT_ref)[:n]
    s2 = jnp.concatenate([slo_ref[...][:n], shi_ref[...][:n]], axis=1)
    c2 = jnp.concatenate([clo_ref[...][:n], chi_ref[...][:n]], axis=1)
    g2out = dinv * (s2 + c2) + b2_ref[...].reshape(1, ch)
    h2_ref[...] = _leaky(
        _bn(g2out, g2_ref[...], be2_ref[...], ch) + h1_ref[...])


def _t3b_body(h2_ref, batch_ref, wih_ref, whh_ref, bih_ref, bhh_ref,
              wl1_ref, bl1_ref, wl2_ref, bl2_ref, out_ref, *, n, nb, steps):
    ch = 2 * _F
    x = h2_ref[...]
    batch = batch_ref[...]
    gid = lax.broadcasted_iota(jnp.int32, (n, nb), 1)
    maskb = batch == gid
    wih = wih_ref[...]
    whh = whh_ref[...]
    bias = (bih_ref[...] + bhh_ref[...]).reshape(1, 4 * ch)
    def step(_, carry):
        h, cst, qs = carry
        gates = (lax.dot_general(qs, wih, (((1,), (1,)), ((), ())),
                                 preferred_element_type=jnp.float32)
                 + lax.dot_general(h, whh, (((1,), (1,)), ((), ())),
                                   preferred_element_type=jnp.float32)
                 + bias)
        i_g = jax.nn.sigmoid(gates[:, :ch])
        f_g = jax.nn.sigmoid(gates[:, ch:2 * ch])
        g_g = jnp.tanh(gates[:, 2 * ch:3 * ch])
        o_g = jax.nn.sigmoid(gates[:, 3 * ch:])
        cst = f_g * cst + i_g * g_g
        h = o_g * jnp.tanh(cst)
        p = lax.dot_general(x, h, (((1,), (1,)), ((), ())),
                            preferred_element_type=jnp.float32)
        pm = jnp.where(maskb, p, -1e30)
        m = jnp.max(pm, axis=0, keepdims=True)
        m0 = jnp.where(m > -1e29, m, 0.0)
        ex = jnp.where(maskb, jnp.exp(p - m0), 0.0)
        ssum = jnp.sum(ex, axis=0, keepdims=True)
        a = ex / (ssum + 1e-16)
        r = lax.dot_general(a, x, (((0,), (0,)), ((), ())),
                            preferred_element_type=jnp.float32)
        qs = jnp.concatenate([h, r], axis=1)
        return h, cst, qs

    _, _, qs = lax.fori_loop(
        0, steps, step,
        (jnp.zeros((nb, ch), jnp.float32),
         jnp.zeros((nb, ch), jnp.float32),
         jnp.zeros((nb, 2 * ch), jnp.float32)))
    o1 = _leaky(jnp.dot(qs, wl1_ref[...], preferred_element_type=jnp.float32)
                + bl1_ref[...].reshape(1, ch))
    out_ref[...] = (jnp.dot(o1, wl2_ref[...],
                            preferred_element_type=jnp.float32)
                    + bl2_ref[...].reshape(1, wl2_ref.shape[1]))


def kernel(x, edge_index, batch_idx, edge_weights, W1, b1, W2, b2, g1, be1,
           g2, be2, W_ih, W_hh, b_ih, b_hh, W_l1, b_l1, W_l2, b_l2):
    n, c_in = x.shape
    e = edge_index.shape[1]
    nb = 64
    steps = 10

    blk = _NC * _NS * _K * 8
    ep = ((e + blk - 1) // blk) * blk
    ngr = _NS * _K
    np_ = ((n + ngr - 1) // ngr) * ngr

    pad = ep - e
    src = jnp.concatenate([edge_index[0], jnp.zeros((pad,), jnp.int32)])
    dst = jnp.concatenate([edge_index[1], jnp.zeros((pad,), jnp.int32)])
    w = jnp.concatenate([edge_weights, jnp.zeros((pad,), jnp.float32)])
    src2 = src.reshape(ep // _K, _K)
    dst2 = dst.reshape(ep // _K, _K)
    w2 = w.reshape(ep // _K, _K)
    xp = jnp.concatenate([x, jnp.zeros((np_ - n, c_in), x.dtype)])

    degp = _deg_call(dst2, w2, np_)
    degT = degp.T

    c1lo, c1hi = _tc(
        _t1_body,
        [jax.ShapeDtypeStruct((np_, _F), jnp.float32)] * 2,
        xp, W1, degT)

    s1lo, s1hi = _agg_call(c1lo, c1hi, src2, dst2, w2, np_)

    h1, c2lo, c2hi = _tc(
        functools.partial(_t2_body, n=n, np_=np_),
        [jax.ShapeDtypeStruct((n, 2 * _F), jnp.float32),
         jax.ShapeDtypeStruct((np_, _F), jnp.float32),
         jax.ShapeDtypeStruct((np_, _F), jnp.float32)],
        s1lo, s1hi, c1lo, c1hi, degT, b1, g1, be1, W2)

    s2lo, s2hi = _agg_call(c2lo, c2hi, src2, dst2, w2, np_)

    h2 = _tc(
        functools.partial(_t3a_body, n=n),
        jax.ShapeDtypeStruct((n, 2 * _F), jnp.float32),
        s2lo, s2hi, c2lo, c2hi, degT, b2, g2, be2, h1)

    out = _tc(
        functools.partial(_t3b_body, n=n, nb=nb, steps=steps),
        jax.ShapeDtypeStruct((nb, W_l2.shape[1]), jnp.float32),
        h2, batch_idx[:, None], W_ih, W_hh, b_ih, b_hh,
        W_l1, b_l1, W_l2, b_l2)
    return out


# speedup vs baseline: 9.2685x; 1.1945x over previous
import functools

import jax
import jax.numpy as jnp
from jax import lax
from jax.experimental import pallas as pl
from jax.experimental.pallas import tpu as pltpu
from jax.experimental.pallas import tpu_sc as plsc

_NC = 2
_NS = 16
_L = 16
_K = 128
_F = 128


def _leaky(v):
    return jnp.where(v >= 0, v, 0.01 * v)


def _mesh():
    return plsc.VectorSubcoreMesh(
        core_axis_name="c", subcore_axis_name="s",
        num_cores=_NC, num_subcores=_NS)


def _deg_call(dst2, w2, np_):
    rows2d = dst2.shape[0]
    per_core = rows2d // _NC
    per_tile = per_core // _NS
    stripe = np_ // _NS

    @functools.partial(
        pl.kernel,
        out_type=jax.ShapeDtypeStruct((_NC, np_), jnp.float32),
        mesh=_mesh(),
        scratch_types=[
            pltpu.VMEM((per_tile, _K), jnp.int32),
            pltpu.VMEM((per_tile, _K), jnp.float32),
            pltpu.VMEM((stripe,), jnp.float32),
            pltpu.VMEM_SHARED((np_,), jnp.float32),
        ],
    )
    def deg_kernel(dst_hbm, w_hbm, out_hbm, dstb, wb, zb, dacc):
        c = lax.axis_index("c")
        s = lax.axis_index("s")
        base = c * per_core + s * per_tile
        pltpu.sync_copy(dst_hbm.at[pl.ds(base, per_tile)], dstb)
        pltpu.sync_copy(w_hbm.at[pl.ds(base, per_tile)], wb)

        def zero_body(i, carry):
            zb[pl.ds(i * _L, _L)] = jnp.zeros((_L,), jnp.float32)
            return carry

        lax.fori_loop(0, stripe // _L, zero_body, 0)
        pltpu.sync_copy(zb, dacc.at[pl.ds(s * stripe, stripe)])
        plsc.subcore_barrier()

        def scat_body(j, carry):
            pltpu.sync_copy(wb.at[j], dacc.at[dstb.at[j]], add=True)
            return carry

        lax.fori_loop(0, per_tile, scat_body, 0)
        plsc.subcore_barrier()
        pltpu.sync_copy(dacc.at[pl.ds(s * stripe, stripe)],
                        out_hbm.at[c, pl.ds(s * stripe, stripe)])

    return deg_kernel(dst2, w2)


def _agg_call(c_lo, c_hi, src2, dst2, w2, np_):
    rows2d = src2.shape[0]
    per_tile = rows2d // _NS
    stripe = np_ // _NS

    @functools.partial(
        pl.kernel,
        out_type=[jax.ShapeDtypeStruct((np_, _F), jnp.float32),
                  jax.ShapeDtypeStruct((np_, _F), jnp.float32)],
        mesh=_mesh(),
        scratch_types=[
            pltpu.VMEM((8, _K), jnp.int32),
            pltpu.VMEM((8, _K), jnp.int32),
            pltpu.VMEM((8, _K), jnp.float32),
            pltpu.VMEM((_K, _F), jnp.float32),
            pltpu.VMEM((_K, _F), jnp.float32),
            pltpu.VMEM_SHARED((np_, _F), jnp.float32),
            pltpu.SemaphoreType.DMA,
            pltpu.SemaphoreType.DMA,
            pltpu.SemaphoreType.DMA,
            pltpu.SemaphoreType.DMA,
        ],
    )
    def agg_kernel(clo_hbm, chi_hbm, src_hbm, dst_hbm, w_hbm,
                   olo_hbm, ohi_hbm, srcb, dstb, wb, rows0, rows1, acc,
                   gs0, gs1, ss0, ss1):
        c = lax.axis_index("c")
        s = lax.axis_index("s")
        rows = (rows0, rows1)
        gsem = (gs0, gs1)
        ssem = (ss0, ss1)

        def zero_row(i, carry):
            for g in range(_F // _L):
                rows0[i, pl.ds(g * _L, _L)] = jnp.zeros((_L,), jnp.float32)
            return carry

        lax.fori_loop(0, _K, zero_row, 0)
        for b in range(stripe // _K):
            pltpu.sync_copy(rows0, acc.at[pl.ds(s * stripe + b * _K, _K)])
        plsc.subcore_barrier()

        def run(chbm, ohbm):
            def scale(buf, wrow):
                def scale16(k16, kc):
                    wv = wb[wrow, pl.ds(k16 * _L, _L)]
                    for i in range(_L):
                        wk = wv[i]
                        for g in range(_F // _L):
                            sl = pl.ds(g * _L, _L)
                            buf[k16 * _L + i, sl] = buf[k16 * _L + i, sl] * wk
                    return kc

                lax.fori_loop(0, _K // _L, scale16, 0)

            def group(jj, carry):
                base = s * per_tile + jj * 8
                pltpu.sync_copy(src_hbm.at[pl.ds(base, 8)], srcb)
                pltpu.sync_copy(dst_hbm.at[pl.ds(base, 8)], dstb)
                pltpu.sync_copy(w_hbm.at[pl.ds(base, 8)], wb)
                g_desc = [None, None]
                s_desc = [None, None]
                g_desc[0] = pltpu.async_copy(
                    chbm.at[srcb.at[0]], rows[0], gsem[0])
                for b in range(8):
                    cur, nxt = b % 2, (b + 1) % 2
                    if b < 7:
                        if b >= 1:
                            s_desc[nxt].wait()
                        g_desc[nxt] = pltpu.async_copy(
                            chbm.at[srcb.at[b + 1]], rows[nxt], gsem[nxt])
                    g_desc[cur].wait()
                    scale(rows[cur], b)
                    s_desc[cur] = pltpu.async_copy(
                        rows[cur], acc.at[dstb.at[b]], ssem[cur], add=True)
                s_desc[0].wait()
                s_desc[1].wait()
                return carry

            lax.fori_loop(0, per_tile // 8, group, 0)
            plsc.subcore_barrier()
            pltpu.sync_copy(acc.at[pl.ds(s * stripe, stripe)],
                            ohbm.at[pl.ds(s * stripe, stripe)])

        @pl.when(c == 0)
        def _():
            run(clo_hbm, olo_hbm)

        @pl.when(c == 1)
        def _():
            run(chi_hbm, ohi_hbm)

    return agg_kernel(c_lo, c_hi, src2, dst2, w2)


def _tc(body, out_shape, *args):
    return pl.pallas_call(
        body,
        out_shape=out_shape,
        compiler_params=pltpu.CompilerParams(
            vmem_limit_bytes=128 * 1024 * 1024),
    )(*args)


def _dinv_from(degT_ref):
    degT = degT_ref[...]
    deg = degT[:, 0:1] + degT[:, 1:2] + 1.0
    return jnp.where(deg > 0, lax.rsqrt(deg), 0.0)


def _t1_body(x_ref, w1_ref, degT_ref, clo_ref, chi_ref):
    dinv = _dinv_from(degT_ref)
    h = jnp.dot(x_ref[...], w1_ref[...], preferred_element_type=jnp.float32)
    cmat = h * dinv
    clo_ref[...] = cmat[:, :_F]
    chi_ref[...] = cmat[:, _F:]


def _bn(z, g, be, n):
    m = jnp.mean(z, axis=0, keepdims=True)
    v = jnp.mean((z - m) ** 2, axis=0, keepdims=True)
    return (z - m) * lax.rsqrt(v + 1e-5) * g.reshape(1, n) + be.reshape(1, n)


def _t2_body(slo_ref, shi_ref, clo_ref, chi_ref, degT_ref, b1_ref, g1_ref,
             be1_ref, w2_ref, h1_ref, c2lo_ref, c2hi_ref, *, n, np_):
    ch = 2 * _F
    dinv = _dinv_from(degT_ref)[:n]
    s1 = jnp.concatenate([slo_ref[...][:n], shi_ref[...][:n]], axis=1)
    c1 = jnp.concatenate([clo_ref[...][:n], chi_ref[...][:n]], axis=1)
    g1out = dinv * (s1 + c1) + b1_ref[...].reshape(1, ch)
    h1 = _leaky(_bn(g1out, g1_ref[...], be1_ref[...], ch))
    h1_ref[...] = h1
    c2 = dinv * jnp.dot(h1, w2_ref[...], preferred_element_type=jnp.float32)
    c2p = jnp.concatenate(
        [c2, jnp.zeros((np_ - n, ch), jnp.float32)], axis=0)
    c2lo_ref[...] = c2p[:, :_F]
    c2hi_ref[...] = c2p[:, _F:]


def _t3a_body(slo_ref, shi_ref, clo_ref, chi_ref, degT_ref, b2_ref, g2_ref,
              be2_ref, h1_ref, h2_ref, *, n):
    ch = 2 * _F
    dinv = _dinv_from(degT_ref)[:n]
    s2 = jnp.concatenate([slo_ref[...][:n], shi_ref[...][:n]], axis=1)
    c2 = jnp.concatenate([clo_ref[...][:n], chi_ref[...][:n]], axis=1)
    g2out = dinv * (s2 + c2) + b2_ref[...].reshape(1, ch)
    h2_ref[...] = _leaky(
        _bn(g2out, g2_ref[...], be2_ref[...], ch) + h1_ref[...])


def _t3b_body(h2_ref, batch_ref, wih_ref, whh_ref, bih_ref, bhh_ref,
              wl1_ref, bl1_ref, wl2_ref, bl2_ref, out_ref, *, n, nb, steps):
    ch = 2 * _F
    x = h2_ref[...]
    batch = batch_ref[...]
    gid = lax.broadcasted_iota(jnp.int32, (n, nb), 1)
    maskb = batch == gid
    wih = wih_ref[...]
    whh = whh_ref[...]
    bias = (bih_ref[...] + bhh_ref[...]).reshape(1, 4 * ch)
    def step(_, carry):
        h, cst, qs = carry
        gates = (lax.dot_general(qs, wih, (((1,), (1,)), ((), ())),
                                 preferred_element_type=jnp.float32)
                 + lax.dot_general(h, whh, (((1,), (1,)), ((), ())),
                                   preferred_element_type=jnp.float32)
                 + bias)
        i_g = jax.nn.sigmoid(gates[:, :ch])
        f_g = jax.nn.sigmoid(gates[:, ch:2 * ch])
        g_g = jnp.tanh(gates[:, 2 * ch:3 * ch])
        o_g = jax.nn.sigmoid(gates[:, 3 * ch:])
        cst = f_g * cst + i_g * g_g
        h = o_g * jnp.tanh(cst)
        p = lax.dot_general(x, h, (((1,), (1,)), ((), ())),
                            preferred_element_type=jnp.float32)
        pm = jnp.where(maskb, p, -1e30)
        m = jnp.max(pm, axis=0, keepdims=True)
        m0 = jnp.where(m > -1e29, m, 0.0)
        ex = jnp.where(maskb, jnp.exp(p - m0), 0.0)
        ssum = jnp.sum(ex, axis=0, keepdims=True)
        a = ex / (ssum + 1e-16)
        r = lax.dot_general(a, x, (((0,), (0,)), ((), ())),
                            preferred_element_type=jnp.float32)
        qs = jnp.concatenate([h, r], axis=1)
        return h, cst, qs

    _, _, qs = lax.fori_loop(
        0, steps, step,
        (jnp.zeros((nb, ch), jnp.float32),
         jnp.zeros((nb, ch), jnp.float32),
         jnp.zeros((nb, 2 * ch), jnp.float32)))
    o1 = _leaky(jnp.dot(qs, wl1_ref[...], preferred_element_type=jnp.float32)
                + bl1_ref[...].reshape(1, ch))
    out_ref[...] = (jnp.dot(o1, wl2_ref[...],
                            preferred_element_type=jnp.float32)
                    + bl2_ref[...].reshape(1, wl2_ref.shape[1]))


def kernel(x, edge_index, batch_idx, edge_weights, W1, b1, W2, b2, g1, be1,
           g2, be2, W_ih, W_hh, b_ih, b_hh, W_l1, b_l1, W_l2, b_l2):
    n, c_in = x.shape
    e = edge_index.shape[1]
    nb = 64
    steps = 10

    blk = _NC * _NS * _K * 8
    ep = ((e + blk - 1) // blk) * blk
    ngr = _NS * _K
    np_ = ((n + ngr - 1) // ngr) * ngr

    pad = ep - e
    src = jnp.concatenate([edge_index[0], jnp.zeros((pad,), jnp.int32)])
    dst = jnp.concatenate([edge_index[1], jnp.zeros((pad,), jnp.int32)])
    w = jnp.concatenate([edge_weights, jnp.zeros((pad,), jnp.float32)])
    src2 = src.reshape(ep // _K, _K)
    dst2 = dst.reshape(ep // _K, _K)
    w2 = w.reshape(ep // _K, _K)
    xp = jnp.concatenate([x, jnp.zeros((np_ - n, c_in), x.dtype)])

    degp = _deg_call(dst2, w2, np_)
    degT = degp.T

    c1lo, c1hi = _tc(
        _t1_body,
        [jax.ShapeDtypeStruct((np_, _F), jnp.float32)] * 2,
        xp, W1, degT)

    s1lo, s1hi = _agg_call(c1lo, c1hi, src2, dst2, w2, np_)

    h1, c2lo, c2hi = _tc(
        functools.partial(_t2_body, n=n, np_=np_),
        [jax.ShapeDtypeStruct((n, 2 * _F), jnp.float32),
         jax.ShapeDtypeStruct((np_, _F), jnp.float32),
         jax.ShapeDtypeStruct((np_, _F), jnp.float32)],
        s1lo, s1hi, c1lo, c1hi, degT, b1, g1, be1, W2)

    s2lo, s2hi = _agg_call(c2lo, c2hi, src2, dst2, w2, np_)

    h2 = _tc(
        functools.partial(_t3a_body, n=n),
        jax.ShapeDtypeStruct((n, 2 * _F), jnp.float32),
        s2lo, s2hi, c2lo, c2hi, degT, b2, g2, be2, h1)

    out = _tc(
        functools.partial(_t3b_body, n=n, nb=nb, steps=steps),
        jax.ShapeDtypeStruct((nb, W_l2.shape[1]), jnp.float32),
        h2, batch_idx[:, None], W_ih, W_hh, b_ih, b_hh,
        W_l1, b_l1, W_l2, b_l2)
    return out


# speedup vs baseline: 10.3114x; 1.1125x over previous
import functools

import jax
import jax.numpy as jnp
from jax import lax
from jax.experimental import pallas as pl
from jax.experimental.pallas import tpu as pltpu
from jax.experimental.pallas import tpu_sc as plsc

_NC = 2
_NS = 16
_L = 16
_K = 128
_F = 128


def _leaky(v):
    return jnp.where(v >= 0, v, 0.01 * v)


def _mesh():
    return plsc.VectorSubcoreMesh(
        core_axis_name="c", subcore_axis_name="s",
        num_cores=_NC, num_subcores=_NS)


def _deg_call(dst2, w2, np_):
    rows2d = dst2.shape[0]
    per_core = rows2d // _NC
    per_tile = per_core // _NS
    stripe = np_ // _NS

    @functools.partial(
        pl.kernel,
        out_type=jax.ShapeDtypeStruct((_NC, np_), jnp.float32),
        mesh=_mesh(),
        scratch_types=[
            pltpu.VMEM((per_tile, _K), jnp.int32),
            pltpu.VMEM((per_tile, _K), jnp.float32),
            pltpu.VMEM((stripe,), jnp.float32),
            pltpu.VMEM_SHARED((np_,), jnp.float32),
        ],
    )
    def deg_kernel(dst_hbm, w_hbm, out_hbm, dstb, wb, zb, dacc):
        c = lax.axis_index("c")
        s = lax.axis_index("s")
        base = c * per_core + s * per_tile
        pltpu.sync_copy(dst_hbm.at[pl.ds(base, per_tile)], dstb)
        pltpu.sync_copy(w_hbm.at[pl.ds(base, per_tile)], wb)

        def zero_body(i, carry):
            zb[pl.ds(i * _L, _L)] = jnp.zeros((_L,), jnp.float32)
            return carry

        lax.fori_loop(0, stripe // _L, zero_body, 0)
        pltpu.sync_copy(zb, dacc.at[pl.ds(s * stripe, stripe)])
        plsc.subcore_barrier()

        def scat_body(j, carry):
            pltpu.sync_copy(wb.at[j], dacc.at[dstb.at[j]], add=True)
            return carry

        lax.fori_loop(0, per_tile, scat_body, 0)
        plsc.subcore_barrier()
        pltpu.sync_copy(dacc.at[pl.ds(s * stripe, stripe)],
                        out_hbm.at[c, pl.ds(s * stripe, stripe)])

    return deg_kernel(dst2, w2)


def _agg_call(c_lo, c_hi, sd, w2, np_):
    rows2d = sd.shape[0] // 2
    per_tile = rows2d // _NS
    stripe = np_ // _NS

    @functools.partial(
        pl.kernel,
        out_type=[jax.ShapeDtypeStruct((np_, _F), jnp.float32),
                  jax.ShapeDtypeStruct((np_, _F), jnp.float32)],
        mesh=_mesh(),
        scratch_types=[
            pltpu.VMEM((16, _K), jnp.int32),
            pltpu.VMEM((8, _K), jnp.float32),
            pltpu.VMEM((_K, _F), jnp.float32),
            pltpu.VMEM((_K, _F), jnp.float32),
            pltpu.VMEM_SHARED((np_, _F), jnp.float32),
            pltpu.SemaphoreType.DMA,
            pltpu.SemaphoreType.DMA,
            pltpu.SemaphoreType.DMA,
            pltpu.SemaphoreType.DMA,
        ],
    )
    def agg_kernel(clo_hbm, chi_hbm, sd_hbm, w_hbm,
                   olo_hbm, ohi_hbm, sdb, wb, rows0, rows1, acc,
                   gs0, gs1, ss0, ss1):
        c = lax.axis_index("c")
        s = lax.axis_index("s")
        rows = (rows0, rows1)
        gsem = (gs0, gs1)
        ssem = (ss0, ss1)

        def zero_row(i, carry):
            for g in range(_F // _L):
                rows0[i, pl.ds(g * _L, _L)] = jnp.zeros((_L,), jnp.float32)
            return carry

        lax.fori_loop(0, _K, zero_row, 0)
        for b in range(stripe // _K):
            pltpu.sync_copy(rows0, acc.at[pl.ds(s * stripe + b * _K, _K)])
        plsc.subcore_barrier()

        def run(chbm, ohbm):
            def scale(buf, wrow):
                def scale16(k16, kc):
                    wv = wb[wrow, pl.ds(k16 * _L, _L)]
                    for i in range(_L):
                        wk = wv[i]
                        for g in range(_F // _L):
                            sl = pl.ds(g * _L, _L)
                            buf[k16 * _L + i, sl] = buf[k16 * _L + i, sl] * wk
                    return kc

                lax.fori_loop(0, _K // _L, scale16, 0)

            def group(jj, carry):
                gbase = s * per_tile + jj * 8
                pltpu.sync_copy(sd_hbm.at[pl.ds(2 * gbase, 16)], sdb)
                pltpu.sync_copy(w_hbm.at[pl.ds(gbase, 8)], wb)
                g_desc = [None, None]
                s_desc = [None, None]
                g_desc[0] = pltpu.async_copy(
                    chbm.at[sdb.at[0]], rows[0], gsem[0])
                for b in range(8):
                    cur, nxt = b % 2, (b + 1) % 2
                    if b < 7:
                        if b >= 1:
                            s_desc[nxt].wait()
                        g_desc[nxt] = pltpu.async_copy(
                            chbm.at[sdb.at[2 * b + 2]], rows[nxt],
                            gsem[nxt])
                    g_desc[cur].wait()
                    scale(rows[cur], b)
                    s_desc[cur] = pltpu.async_copy(
                        rows[cur], acc.at[sdb.at[2 * b + 1]], ssem[cur],
                        add=True)
                s_desc[0].wait()
                s_desc[1].wait()
                return carry

            lax.fori_loop(0, per_tile // 8, group, 0)
            plsc.subcore_barrier()
            pltpu.sync_copy(acc.at[pl.ds(s * stripe, stripe)],
                            ohbm.at[pl.ds(s * stripe, stripe)])

        @pl.when(c == 0)
        def _():
            run(clo_hbm, olo_hbm)

        @pl.when(c == 1)
        def _():
            run(chi_hbm, ohi_hbm)

    return agg_kernel(c_lo, c_hi, sd, w2)


def _tc(body, out_shape, *args):
    return pl.pallas_call(
        body,
        out_shape=out_shape,
        compiler_params=pltpu.CompilerParams(
            vmem_limit_bytes=128 * 1024 * 1024),
    )(*args)


def _dinv_from(degT_ref):
    degT = degT_ref[...]
    deg = degT[:, 0:1] + degT[:, 1:2] + 1.0
    return jnp.where(deg > 0, lax.rsqrt(deg), 0.0)


def _t1_body(x_ref, w1_ref, degT_ref, clo_ref, chi_ref):
    dinv = _dinv_from(degT_ref)
    h = jnp.dot(x_ref[...], w1_ref[...], preferred_element_type=jnp.float32)
    cmat = h * dinv
    clo_ref[...] = cmat[:, :_F]
    chi_ref[...] = cmat[:, _F:]


def _bn(z, g, be, n):
    m = jnp.mean(z, axis=0, keepdims=True)
    v = jnp.mean((z - m) ** 2, axis=0, keepdims=True)
    return (z - m) * lax.rsqrt(v + 1e-5) * g.reshape(1, n) + be.reshape(1, n)


def _t2_body(slo_ref, shi_ref, clo_ref, chi_ref, degT_ref, b1_ref, g1_ref,
             be1_ref, w2_ref, h1_ref, c2lo_ref, c2hi_ref, *, n, np_):
    ch = 2 * _F
    dinv = _dinv_from(degT_ref)[:n]
    s1 = jnp.concatenate([slo_ref[...][:n], shi_ref[...][:n]], axis=1)
    c1 = jnp.concatenate([clo_ref[...][:n], chi_ref[...][:n]], axis=1)
    g1out = dinv * (s1 + c1) + b1_ref[...].reshape(1, ch)
    h1 = _leaky(_bn(g1out, g1_ref[...], be1_ref[...], ch))
    h1_ref[...] = h1
    c2 = dinv * jnp.dot(h1, w2_ref[...], preferred_element_type=jnp.float32)
    c2p = jnp.concatenate(
        [c2, jnp.zeros((np_ - n, ch), jnp.float32)], axis=0)
    c2lo_ref[...] = c2p[:, :_F]
    c2hi_ref[...] = c2p[:, _F:]


def _t3a_body(slo_ref, shi_ref, clo_ref, chi_ref, degT_ref, b2_ref, g2_ref,
              be2_ref, h1_ref, h2_ref, *, n):
    ch = 2 * _F
    dinv = _dinv_from(degT_ref)[:n]
    s2 = jnp.concatenate([slo_ref[...][:n], shi_ref[...][:n]], axis=1)
    c2 = jnp.concatenate([clo_ref[...][:n], chi_ref[...][:n]], axis=1)
    g2out = dinv * (s2 + c2) + b2_ref[...].reshape(1, ch)
    h2_ref[...] = _leaky(
        _bn(g2out, g2_ref[...], be2_ref[...], ch) + h1_ref[...])


def _t3b_body(h2_ref, batch_ref, wih_ref, whh_ref, bih_ref, bhh_ref,
              wl1_ref, bl1_ref, wl2_ref, bl2_ref, out_ref, *, n, nb, steps):
    ch = 2 * _F
    x = h2_ref[...]
    batch = batch_ref[...]
    gid = lax.broadcasted_iota(jnp.int32, (n, nb), 1)
    maskb = batch == gid
    wih = wih_ref[...]
    whh = whh_ref[...]
    bias = (bih_ref[...] + bhh_ref[...]).reshape(1, 4 * ch)
    def step(_, carry):
        h, cst, qs = carry
        gates = (lax.dot_general(qs, wih, (((1,), (1,)), ((), ())),
                                 preferred_element_type=jnp.float32)
                 + lax.dot_general(h, whh, (((1,), (1,)), ((), ())),
                                   preferred_element_type=jnp.float32)
                 + bias)
        i_g = jax.nn.sigmoid(gates[:, :ch])
        f_g = jax.nn.sigmoid(gates[:, ch:2 * ch])
        g_g = jnp.tanh(gates[:, 2 * ch:3 * ch])
        o_g = jax.nn.sigmoid(gates[:, 3 * ch:])
        cst = f_g * cst + i_g * g_g
        h = o_g * jnp.tanh(cst)
        p = lax.dot_general(x, h, (((1,), (1,)), ((), ())),
                            preferred_element_type=jnp.float32)
        pm = jnp.where(maskb, p, -1e30)
        m = jnp.max(pm, axis=0, keepdims=True)
        m0 = jnp.where(m > -1e29, m, 0.0)
        ex = jnp.where(maskb, jnp.exp(p - m0), 0.0)
        ssum = jnp.sum(ex, axis=0, keepdims=True)
        a = ex / (ssum + 1e-16)
        r = lax.dot_general(a, x, (((0,), (0,)), ((), ())),
                            preferred_element_type=jnp.float32)
        qs = jnp.concatenate([h, r], axis=1)
        return h, cst, qs

    _, _, qs = lax.fori_loop(
        0, steps, step,
        (jnp.zeros((nb, ch), jnp.float32),
         jnp.zeros((nb, ch), jnp.float32),
         jnp.zeros((nb, 2 * ch), jnp.float32)))
    o1 = _leaky(jnp.dot(qs, wl1_ref[...], preferred_element_type=jnp.float32)
                + bl1_ref[...].reshape(1, ch))
    out_ref[...] = (jnp.dot(o1, wl2_ref[...],
                            preferred_element_type=jnp.float32)
                    + bl2_ref[...].reshape(1, wl2_ref.shape[1]))


def kernel(x, edge_index, batch_idx, edge_weights, W1, b1, W2, b2, g1, be1,
           g2, be2, W_ih, W_hh, b_ih, b_hh, W_l1, b_l1, W_l2, b_l2):
    n, c_in = x.shape
    e = edge_index.shape[1]
    nb = 64
    steps = 10

    blk = _NC * _NS * _K * 8
    ep = ((e + blk - 1) // blk) * blk
    ngr = _NS * _K
    np_ = ((n + ngr - 1) // ngr) * ngr

    pad = ep - e
    src = jnp.concatenate([edge_index[0], jnp.zeros((pad,), jnp.int32)])
    dst = jnp.concatenate([edge_index[1], jnp.zeros((pad,), jnp.int32)])
    w = jnp.concatenate([edge_weights, jnp.zeros((pad,), jnp.float32)])
    src2 = src.reshape(ep // _K, _K)
    dst2 = dst.reshape(ep // _K, _K)
    w2 = w.reshape(ep // _K, _K)
    sd = jnp.stack([src2, dst2], axis=1).reshape(2 * (ep // _K), _K)
    xp = jnp.concatenate([x, jnp.zeros((np_ - n, c_in), x.dtype)])

    degp = _deg_call(dst2, w2, np_)
    degT = degp.T

    c1lo, c1hi = _tc(
        _t1_body,
        [jax.ShapeDtypeStruct((np_, _F), jnp.float32)] * 2,
        xp, W1, degT)

    s1lo, s1hi = _agg_call(c1lo, c1hi, sd, w2, np_)

    h1, c2lo, c2hi = _tc(
        functools.partial(_t2_body, n=n, np_=np_),
        [jax.ShapeDtypeStruct((n, 2 * _F), jnp.float32),
         jax.ShapeDtypeStruct((np_, _F), jnp.float32),
         jax.ShapeDtypeStruct((np_, _F), jnp.float32)],
        s1lo, s1hi, c1lo, c1hi, degT, b1, g1, be1, W2)

    s2lo, s2hi = _agg_call(c2lo, c2hi, sd, w2, np_)

    h2 = _tc(
        functools.partial(_t3a_body, n=n),
        jax.ShapeDtypeStruct((n, 2 * _F), jnp.float32),
        s2lo, s2hi, c2lo, c2hi, degT, b2, g2, be2, h1)

    out = _tc(
        functools.partial(_t3b_body, n=n, nb=nb, steps=steps),
        jax.ShapeDtypeStruct((nb, W_l2.shape[1]), jnp.float32),
        h2, batch_idx[:, None], W_ih, W_hh, b_ih, b_hh,
        W_l1, b_l1, W_l2, b_l2)
    return out
